# Initial kernel scaffold; baseline (speedup 1.0000x reference)
import jax
import jax.numpy as jnp
from jax.experimental import pallas as pl


def kernel(X, Y):
    raise NotImplementedError("write your pallas kernel here")


# speedup vs baseline: 7.3206x; 7.3206x over previous
import jax
import jax.numpy as jnp
from jax.experimental import pallas as pl
from jax.experimental.pallas import tpu as pltpu

_B, _N, _M, _DIM = 16, 128, 128, 64
_INF = 1000000000.0


def _ldtw_tc_kernel(x_ref, y_ref, out_ref, s_ref):
    lane_n = jax.lax.broadcasted_iota(jnp.int32, (2 * _N, _N), 1)

    for b in range(_B):
        Xb = x_ref[b]
        Yb = y_ref[b]
        x2 = jnp.sum(Xb * Xb, axis=1, keepdims=True)
        y2 = jnp.sum(Yb * Yb, axis=1, keepdims=True)
        ones_n = jnp.ones((_N, 1), jnp.float32)
        ones_m = jnp.ones((_M, 1), jnp.float32)
        Aaug = jnp.concatenate([-2.0 * Yb, ones_m, y2], axis=1)
        Baug = jnp.concatenate([Xb, x2, ones_n], axis=1)
        Db = jax.lax.dot_general(
            Aaug, Baug,
            (((1,), (1,)), ((), ())),
            preferred_element_type=jnp.float32,
            precision=jax.lax.Precision.HIGHEST,
        )

        S = jnp.concatenate(
            [Db, jnp.full((2 * _N - _M, _N), _INF, jnp.float32)], axis=0)
        for k in range(7):
            bit = 1 << k
            rolled = jnp.concatenate([S[-bit:], S[:-bit]], axis=0)
            S = jnp.where((lane_n & bit) != 0, rolled, S)
        s_ref[:, b, :] = S

    lane2 = jax.lax.broadcasted_iota(jnp.int32, (_B, _N), 1)

    def shift1(a):
        return jnp.where(lane2 == 0, _INF,
                         jnp.concatenate([a[:, -1:], a[:, :-1]], axis=1))

    a1 = s_ref[0] + jnp.where(lane2 == 0, 0.0, _INF)
    a2 = jnp.full((_B, _N), _INF, jnp.float32)

    def step(d, carry):
        a2c, a1c = carry
        t = s_ref[d]
        a0 = t + jnp.minimum(jnp.minimum(a1c, shift1(a1c)), shift1(a2c))
        return (a1c, a0)

    _, a1 = jax.lax.fori_loop(1, 2 * _N - 1, step, (a2, a1))
    out_ref[...] = a1


def kernel(X, Y):
    out = pl.pallas_call(
        _ldtw_tc_kernel,
        out_shape=jax.ShapeDtypeStruct((_B, _N), jnp.float32),
        scratch_shapes=[pltpu.VMEM((2 * _N, _B, _N), jnp.float32)],
    )(X, Y)
    return out[:, _N - 1]


# speedup vs baseline: 7.4596x; 1.0190x over previous
import jax
import jax.numpy as jnp
from jax.experimental import pallas as pl
from jax.experimental.pallas import tpu as pltpu

_B, _N, _M, _DIM = 16, 128, 128, 64
_INF = 1000000000.0


def _ldtw_tc_kernel(x_ref, y_ref, out_ref, s_ref):
    lane_n = jax.lax.broadcasted_iota(jnp.int32, (2 * _N, _N), 1)

    for b in range(_B):
        Xb = x_ref[b]
        Yb = y_ref[b]
        x2 = jnp.sum(Xb * Xb, axis=1, keepdims=True)
        y2 = jnp.sum(Yb * Yb, axis=1, keepdims=True)
        ones_n = jnp.ones((_N, 1), jnp.float32)
        ones_m = jnp.ones((_M, 1), jnp.float32)
        Aaug = jnp.concatenate([-2.0 * Yb, ones_m, y2], axis=1)
        Baug = jnp.concatenate([Xb, x2, ones_n], axis=1)
        Db = jax.lax.dot_general(
            Aaug, Baug,
            (((1,), (1,)), ((), ())),
            preferred_element_type=jnp.float32,
            precision=jax.lax.Precision.HIGHEST,
        )

        S = jnp.concatenate(
            [Db, jnp.full((2 * _N - _M, _N), _INF, jnp.float32)], axis=0)
        for k in range(7):
            bit = 1 << k
            rolled = jnp.concatenate([S[-bit:], S[:-bit]], axis=0)
            S = jnp.where((lane_n & bit) != 0, rolled, S)
        s_ref[:, b, :] = S

    lane2 = jax.lax.broadcasted_iota(jnp.int32, (_B, _N), 1)

    def shift1(a):
        return jnp.where(lane2 == 0, _INF,
                         jnp.concatenate([a[:, -1:], a[:, :-1]], axis=1))

    a1 = s_ref[0] + jnp.where(lane2 == 0, 0.0, _INF)
    a2 = jnp.full((_B, _N), _INF, jnp.float32)

    def onestep(t, a2c, a1c):
        return t + jnp.minimum(a1c, shift1(jnp.minimum(a1c, a2c)))

    def step2(k, carry):
        a2c, a1c = carry
        ts = s_ref[pl.ds(1 + 2 * k, 2)]
        a0 = onestep(ts[0], a2c, a1c)
        am = onestep(ts[1], a1c, a0)
        return (a0, am)

    _, a1 = jax.lax.fori_loop(0, _N - 1, step2, (a2, a1))
    out_ref[...] = a1


def kernel(X, Y):
    out = pl.pallas_call(
        _ldtw_tc_kernel,
        out_shape=jax.ShapeDtypeStruct((_B, _N), jnp.float32),
        scratch_shapes=[pltpu.VMEM((2 * _N, _B, _N), jnp.float32)],
    )(X, Y)
    return out[:, _N - 1]
